# Initial kernel scaffold; baseline (speedup 1.0000x reference)
import jax
import jax.numpy as jnp
from jax.experimental import pallas as pl


def kernel(vocab_id, table, W, b):
    raise NotImplementedError("write your pallas kernel here")


# speedup vs baseline: 1.3198x; 1.3198x over previous
import dataclasses
import functools

import jax
import jax.numpy as jnp
from jax import lax
from jax.experimental import pallas as pl
from jax.experimental.pallas import tpu as pltpu
from jax.experimental.pallas import tpu_sc as plsc

B = 4096
L = 200
D = 32
NW = 32
BPW = B // NW
NG = BPW // 16


def _rsqrt(x):
    i = lax.bitcast_convert_type(x, jnp.int32)
    i = jnp.int32(0x5F3759DF) - (i >> 1)
    y = lax.bitcast_convert_type(i, jnp.float32)
    xh = x * 0.5
    for _ in range(3):
        y = y * (1.5 - xh * y * y)
    return y


def _body(vid_hbm, tab_hbm, w_hbm, b_hbm, out_hbm,
          idx_v, w_v, rows_v, acc_v, bias_v, sem):
    wid = lax.axis_index("s") * 2 + lax.axis_index("c")
    base = wid * BPW

    pltpu.sync_copy(vid_hbm.at[:, pl.ds(base, BPW)], idx_v)
    pltpu.sync_copy(w_hbm, w_v)
    pltpu.sync_copy(b_hbm, bias_v)

    zeros = jnp.zeros((16,), jnp.float32)
    for g in range(NG):
        acc_v[pl.ds(g * 16, 16)] = zeros

    iota = lax.iota(jnp.int32, 16)
    row_idx = [iota + g * 16 for g in range(NG)]

    @pl.loop(0, L)
    def _(l):
        pltpu.async_copy(tab_hbm.at[idx_v.at[l]], rows_v, sem).wait()

        lvec = jnp.broadcast_to(l, (16,)).astype(jnp.int32)
        dot = [zeros] * NG
        nrm = [zeros] * NG
        for d in range(D):
            dvec = jnp.full((16,), d, jnp.int32)
            wd = plsc.load_gather(w_v, [lvec, dvec])
            for g in range(NG):
                col = plsc.load_gather(rows_v, [row_idx[g], dvec])
                dot[g] = dot[g] + col * wd
                nrm[g] = nrm[g] + col * col
        for g in range(NG):
            scale = jnp.minimum(_rsqrt(nrm[g]), 1.0)
            plsc.addupdate(acc_v.at[pl.ds(g * 16, 16)], dot[g] * scale)

    bias = bias_v[...]
    for g in range(NG):
        x = acc_v[pl.ds(g * 16, 16)] + bias
        acc_v[pl.ds(g * 16, 16)] = 1.0 / (1.0 + jnp.exp(-x))
    pltpu.sync_copy(acc_v, out_hbm.at[pl.ds(base, BPW)])


def kernel(vocab_id, table, W, b):
    vid_t = vocab_id.T.astype(jnp.int32)
    w2 = W.reshape(L, D).astype(jnp.float32)
    bias16 = jnp.broadcast_to(b.astype(jnp.float32), (16,))

    mesh = plsc.VectorSubcoreMesh(
        core_axis_name="c", subcore_axis_name="s",
        num_cores=2, num_subcores=16,
    )
    cp = pltpu.CompilerParams()
    for field, val in (("needs_layout_passes", False),
                       ("use_tc_tiling_on_sc", False)):
        if field in pltpu.CompilerParams.__dataclass_fields__:
            cp = dataclasses.replace(cp, **{field: val})
    run = pl.kernel(
        _body,
        out_type=jax.ShapeDtypeStruct((B,), jnp.float32),
        mesh=mesh,
        compiler_params=cp,
        scratch_types=[
            pltpu.VMEM((L, BPW), jnp.int32),
            pltpu.VMEM((L, D), jnp.float32),
            pltpu.VMEM((BPW, D), jnp.float32),
            pltpu.VMEM((BPW,), jnp.float32),
            pltpu.VMEM((16,), jnp.float32),
            pltpu.SemaphoreType.DMA,
        ],
    )
    return run(vid_t, table, w2, bias16)


# speedup vs baseline: 1.5181x; 1.1502x over previous
import dataclasses
import functools

import jax
import jax.numpy as jnp
from jax import lax
from jax.experimental import pallas as pl
from jax.experimental.pallas import tpu as pltpu
from jax.experimental.pallas import tpu_sc as plsc

B = 4096
L = 200
D = 32
NW = 32
BPW = B // NW
NG = BPW // 16


def _rsqrt(x):
    i = lax.bitcast_convert_type(x, jnp.int32)
    i = jnp.int32(0x5F3759DF) - (i >> 1)
    y = lax.bitcast_convert_type(i, jnp.float32)
    xh = x * 0.5
    for _ in range(3):
        y = y * (1.5 - xh * y * y)
    return y


NBUF = 4


def _body(vid_hbm, tab_hbm, w_hbm, b_hbm, out_hbm,
          idx_v, w_v, rows_v, acc_v, bias_v, sems):
    wid = lax.axis_index("s") * 2 + lax.axis_index("c")
    base = wid * BPW

    pltpu.sync_copy(vid_hbm.at[:, pl.ds(base, BPW)], idx_v)
    pltpu.sync_copy(w_hbm, w_v)
    pltpu.sync_copy(b_hbm, bias_v)

    zeros = jnp.zeros((16,), jnp.float32)
    for g in range(NG):
        acc_v[pl.ds(g * 16, 16)] = zeros

    iota = lax.iota(jnp.int32, 16)
    row_idx = [iota + g * 16 for g in range(NG)]

    def issue(l, j):
        pltpu.make_async_copy(
            tab_hbm.at[idx_v.at[l]], rows_v[j], sems[j]).start()

    def compute(l, j):
        pltpu.make_async_copy(
            tab_hbm.at[idx_v.at[l]], rows_v[j], sems[j]).wait()
        lvec = jnp.broadcast_to(l, (16,)).astype(jnp.int32)
        dot = [zeros] * NG
        nrm = [zeros] * NG
        for d in range(D):
            dvec = jnp.full((16,), d, jnp.int32)
            wd = plsc.load_gather(w_v, [lvec, dvec])
            for g in range(NG):
                col = plsc.load_gather(rows_v[j], [row_idx[g], dvec])
                dot[g] = dot[g] + col * wd
                nrm[g] = nrm[g] + col * col
        for g in range(NG):
            scale = jnp.minimum(_rsqrt(nrm[g]), 1.0)
            plsc.addupdate(acc_v.at[pl.ds(g * 16, 16)], dot[g] * scale)

    for j in range(NBUF):
        issue(j, j)

    @pl.loop(0, L, step=NBUF)
    def _(l):
        for j in range(NBUF):
            compute(l + j, j)

            @pl.when(l + j + NBUF < L)
            def _():
                issue(l + j + NBUF, j)

    bias = bias_v[...]
    for g in range(NG):
        x = acc_v[pl.ds(g * 16, 16)] + bias
        acc_v[pl.ds(g * 16, 16)] = 1.0 / (1.0 + jnp.exp(-x))
    pltpu.sync_copy(acc_v, out_hbm.at[pl.ds(base, BPW)])


def kernel(vocab_id, table, W, b):
    vid_t = vocab_id.T.astype(jnp.int32)
    w2 = W.reshape(L, D).astype(jnp.float32)
    bias16 = jnp.broadcast_to(b.astype(jnp.float32), (16,))

    mesh = plsc.VectorSubcoreMesh(
        core_axis_name="c", subcore_axis_name="s",
        num_cores=2, num_subcores=16,
    )
    cp = pltpu.CompilerParams()
    for field, val in (("needs_layout_passes", False),
                       ("use_tc_tiling_on_sc", False)):
        if field in pltpu.CompilerParams.__dataclass_fields__:
            cp = dataclasses.replace(cp, **{field: val})
    run = pl.kernel(
        _body,
        out_type=jax.ShapeDtypeStruct((B,), jnp.float32),
        mesh=mesh,
        compiler_params=cp,
        scratch_types=[
            pltpu.VMEM((L, BPW), jnp.int32),
            pltpu.VMEM((L, D), jnp.float32),
            [pltpu.VMEM((BPW, D), jnp.float32)] * NBUF,
            pltpu.VMEM((BPW,), jnp.float32),
            pltpu.VMEM((16,), jnp.float32),
            [pltpu.SemaphoreType.DMA] * NBUF,
        ],
    )
    return run(vid_t, table, w2, bias16)


# speedup vs baseline: 2.0999x; 1.3833x over previous
import dataclasses
import functools

import jax
import jax.numpy as jnp
from jax import lax
from jax.experimental import pallas as pl
from jax.experimental.pallas import tpu as pltpu
from jax.experimental.pallas import tpu_sc as plsc

B = 4096
L = 200
D = 32
NW = 32
BPW = B // NW
NG = BPW // 16


def _rsqrt(x):
    i = lax.bitcast_convert_type(x, jnp.int32)
    i = jnp.int32(0x5F3759DF) - (i >> 1)
    y = lax.bitcast_convert_type(i, jnp.float32)
    xh = x * 0.5
    for _ in range(3):
        y = y * (1.5 - xh * y * y)
    return y


NBUF = 4


def _body(vid_hbm, tab_hbm, w_hbm, b_hbm, out_hbm,
          idx_v, w_v, rows_v, acc_v, bias_v, sems):
    wid = lax.axis_index("s") * 2 + lax.axis_index("c")
    base = wid * BPW

    pltpu.sync_copy(vid_hbm.at[:, pl.ds(base, BPW)], idx_v)
    pltpu.sync_copy(w_hbm, w_v)
    pltpu.sync_copy(b_hbm, bias_v)

    zeros = jnp.zeros((16,), jnp.float32)
    for g in range(NG):
        acc_v[pl.ds(g * 16, 16)] = zeros

    iota = lax.iota(jnp.int32, 16)
    row_idx = [iota + g * 16 for g in range(NG)]

    rot = [(iota + d) % D for d in range(D)]

    def issue(l, j):
        pltpu.make_async_copy(
            tab_hbm.at[idx_v.at[l]], rows_v[j], sems[j]).start()

    def compute(l, j):
        pltpu.make_async_copy(
            tab_hbm.at[idx_v.at[l]], rows_v[j], sems[j]).wait()
        lvec = jnp.broadcast_to(l, (16,)).astype(jnp.int32)
        dot = [zeros] * NG
        nrm = [zeros] * NG
        for d in range(D):
            wd = plsc.load_gather(w_v, [lvec, rot[d]])
            for g in range(NG):
                col = plsc.load_gather(rows_v[j], [row_idx[g], rot[d]])
                dot[g] = dot[g] + col * wd
                nrm[g] = nrm[g] + col * col
        for g in range(NG):
            scale = jnp.minimum(_rsqrt(nrm[g]), 1.0)
            plsc.addupdate(acc_v.at[pl.ds(g * 16, 16)], dot[g] * scale)

    for j in range(NBUF):
        issue(j, j)

    @pl.loop(0, L, step=NBUF)
    def _(l):
        for j in range(NBUF):
            compute(l + j, j)

            @pl.when(l + j + NBUF < L)
            def _():
                issue(l + j + NBUF, j)

    bias = bias_v[...]
    for g in range(NG):
        x = acc_v[pl.ds(g * 16, 16)] + bias
        acc_v[pl.ds(g * 16, 16)] = 1.0 / (1.0 + jnp.exp(-x))
    pltpu.sync_copy(acc_v, out_hbm.at[pl.ds(base, BPW)])


def kernel(vocab_id, table, W, b):
    vid_t = vocab_id.T.astype(jnp.int32)
    w2 = W.reshape(L, D).astype(jnp.float32)
    bias16 = jnp.broadcast_to(b.astype(jnp.float32), (16,))

    mesh = plsc.VectorSubcoreMesh(
        core_axis_name="c", subcore_axis_name="s",
        num_cores=2, num_subcores=16,
    )
    cp = pltpu.CompilerParams()
    for field, val in (("needs_layout_passes", False),
                       ("use_tc_tiling_on_sc", False)):
        if field in pltpu.CompilerParams.__dataclass_fields__:
            cp = dataclasses.replace(cp, **{field: val})
    run = pl.kernel(
        _body,
        out_type=jax.ShapeDtypeStruct((B,), jnp.float32),
        mesh=mesh,
        compiler_params=cp,
        scratch_types=[
            pltpu.VMEM((L, BPW), jnp.int32),
            pltpu.VMEM((L, D), jnp.float32),
            [pltpu.VMEM((BPW, D), jnp.float32)] * NBUF,
            pltpu.VMEM((BPW,), jnp.float32),
            pltpu.VMEM((16,), jnp.float32),
            [pltpu.SemaphoreType.DMA] * NBUF,
        ],
    )
    return run(vid_t, table, w2, bias16)
